# baseline (speedup 1.0000x reference)
import functools

import jax
import jax.numpy as jnp
from jax import lax
from jax.experimental import pallas as pl
from jax.experimental.pallas import tpu as pltpu
from jax.experimental.pallas import tpu_sc as plsc

_TOKENS, _DIN, _DOUT, _E = 4096, 768, 1024, 8
_H = 1536
_BLK = 256
_NW = 32
_TPW = _TOKENS // _NW


def _dot(a, b):
    return jax.lax.dot_general(a, b, (((1,), (0,)), ((), ())),
                               preferred_element_type=jnp.float32)


def _router_body(x_ref, w1_ref, b1_ref, w2_ref, b2_ref, o_ref):
    x = x_ref[...]
    h = jnp.maximum(_dot(x, w1_ref[...]) + b1_ref[...], 0.0)
    s = _dot(h, w2_ref[...]) + b2_ref[...]
    o_ref[...] = jnp.argmax(s, axis=1).astype(jnp.int32)


def _router(x, W1, b1r, w2r, b2r):
    nb = _TOKENS // _BLK
    return pl.pallas_call(
        _router_body,
        grid=(nb,),
        in_specs=[
            pl.BlockSpec((_BLK, _DIN), lambda b: (b, 0)),
            pl.BlockSpec((_DIN, _H), lambda b: (0, 0)),
            pl.BlockSpec((1, _H), lambda b: (0, 0)),
            pl.BlockSpec((_H, _E), lambda b: (0, 0)),
            pl.BlockSpec((1, _E), lambda b: (0, 0)),
        ],
        out_specs=pl.BlockSpec((_BLK,), lambda b: (b,)),
        out_shape=jax.ShapeDtypeStruct((_TOKENS,), jnp.int32),
    )(x, W1, b1r, w2r, b2r)


_GDN = lax.GatherDimensionNumbers(
    offset_dims=(), collapsed_slice_dims=(0,), start_index_map=(0,))


def _take16(v, idx):
    return lax.gather(v, idx[:, None], _GDN, (1,),
                      mode=lax.GatherScatterMode.PROMISE_IN_BOUNDS)


def _splat16(v, j):
    return _take16(v, jnp.full((16,), j, jnp.int32))


def _cumsum16(v, lanes):
    for sh in (1, 2, 4, 8):
        idx = jnp.maximum(lanes - sh, 0)
        v = v + jnp.where(lanes >= sh, _take16(v, idx), 0)
    return v


def _count_slice(ids_v, base, lanes):
    cnts = [jnp.zeros((16,), jnp.int32) for _ in range(_E)]
    for k in range(8):
        v = ids_v[pl.ds(base + 16 * k, 16)]
        for e in range(_E):
            mi = jnp.where(v == e, 1, 0)
            cnts[e] = cnts[e] + _splat16(_cumsum16(mi, lanes), 15)
    vec = jnp.zeros((16,), jnp.int32)
    for e in range(_E):
        vec = jnp.where(lanes == e, cnts[e], vec)
    return vec


def _dispatch_body(ids_hbm, x_hbm, dest_hbm, xs_hbm, starts_hbm,
                   ids2_v, idsw_v, cnt2_v, table_v, dest_v, starts_v,
                   xrows_v, shared_cnt, sem, sem_x, sem_w):
    c = lax.axis_index("c")
    s = lax.axis_index("s")
    w = 16 * c + s
    lanes = lax.iota(jnp.int32, 16)

    cp_x = pltpu.async_copy(x_hbm.at[pl.ds(_TPW * w, _TPW)], xrows_v, sem_x)
    cp_w = pltpu.async_copy(ids_hbm.at[pl.ds(_TPW * w, _TPW)], idsw_v, sem_w)

    pltpu.sync_copy(ids_hbm.at[pl.ds(256 * s, 256)], ids2_v)
    for j in range(2):
        cnt2_v[pl.ds(16 * j, 16)] = _count_slice(ids2_v, 128 * j, lanes)
    pltpu.sync_copy(cnt2_v, shared_cnt.at[pl.ds(32 * s, 32)])
    plsc.subcore_barrier()
    pltpu.sync_copy(shared_cnt, table_v)

    totals = jnp.zeros((16,), jnp.int32)
    prior = jnp.zeros((16,), jnp.int32)
    for r in range(_NW):
        row = table_v[pl.ds(16 * r, 16)]
        totals = totals + row
        prior = prior + row * jnp.where(r < w, 1, 0)
    starts_vec = _cumsum16(totals, lanes) - totals
    base_vec = starts_vec + prior

    @pl.when(w == 0)
    def _():
        starts_v[...] = starts_vec
        pltpu.sync_copy(starts_v, starts_hbm)

    cp_w.wait()
    base_run = [_splat16(base_vec, e) for e in range(_E)]
    for k in range(8):
        v = idsw_v[pl.ds(16 * k, 16)]
        dest = jnp.zeros((16,), jnp.int32)
        for e in range(_E):
            mi = jnp.where(v == e, 1, 0)
            incl = _cumsum16(mi, lanes)
            cand = base_run[e] + incl - 1
            dest = dest + mi * (cand - dest)
            base_run[e] = base_run[e] + _splat16(incl, 15)
        dest_v[pl.ds(16 * k, 16)] = dest
    pltpu.sync_copy(dest_v, dest_hbm.at[pl.ds(_TPW * w, _TPW)])

    cp_x.wait()
    pltpu.async_copy(xrows_v, xs_hbm.at[dest_v], sem).wait()


def _dispatch(ids, x):
    mesh = plsc.VectorSubcoreMesh(core_axis_name="c", subcore_axis_name="s")
    f = pl.kernel(
        _dispatch_body,
        out_type=(
            jax.ShapeDtypeStruct((_TOKENS,), jnp.int32),
            jax.ShapeDtypeStruct((_TOKENS, _DIN), jnp.float32),
            jax.ShapeDtypeStruct((16,), jnp.int32),
        ),
        mesh=mesh,
        scratch_types=[
            pltpu.VMEM((256,), jnp.int32),
            pltpu.VMEM((_TPW,), jnp.int32),
            pltpu.VMEM((32,), jnp.int32),
            pltpu.VMEM((16 * _NW,), jnp.int32),
            pltpu.VMEM((_TPW,), jnp.int32),
            pltpu.VMEM((16,), jnp.int32),
            pltpu.VMEM((_TPW, _DIN), jnp.float32),
            pltpu.VMEM_SHARED((16 * _NW,), jnp.int32),
            pltpu.SemaphoreType.DMA,
            pltpu.SemaphoreType.DMA,
            pltpu.SemaphoreType.DMA,
        ],
    )
    return f(ids, x)


def _segmm_body(s_ref, xs_ref, we_ref, be_ref, o_ref):
    b = pl.program_id(0)
    r0 = b * _BLK
    o_ref[...] = jnp.zeros((_BLK, _DOUT), jnp.float32)
    rowid = r0 + jax.lax.broadcasted_iota(jnp.int32, (_BLK, 1), 0)
    for e in range(_E):
        lo = s_ref[e]
        hi = s_ref[e + 1]
        cond = jnp.logical_and(lo < r0 + _BLK, hi > r0)

        @pl.when(cond)
        def _(e=e, lo=lo, hi=hi):
            xb = xs_ref[...].astype(jnp.bfloat16)
            wb = we_ref[:, e * _DOUT:(e + 1) * _DOUT]
            pe = _dot(xb, wb) + be_ref[:, e * _DOUT:(e + 1) * _DOUT]
            m = jnp.logical_and(rowid >= lo, rowid < hi)
            o_ref[...] += jnp.where(m, pe, 0.0)


def _segmm(starts, xs, We, ber):
    nb = _TOKENS // _BLK
    grid_spec = pltpu.PrefetchScalarGridSpec(
        num_scalar_prefetch=1,
        grid=(nb,),
        in_specs=[
            pl.BlockSpec((_BLK, _DIN), lambda b, s: (b, 0)),
            pl.BlockSpec((_DIN, _E * _DOUT), lambda b, s: (0, 0)),
            pl.BlockSpec((1, _E * _DOUT), lambda b, s: (0, 0)),
        ],
        out_specs=pl.BlockSpec((_BLK, _DOUT), lambda b, s: (b, 0)),
    )
    return pl.pallas_call(
        _segmm_body,
        grid_spec=grid_spec,
        out_shape=jax.ShapeDtypeStruct((_TOKENS, _DOUT), jnp.float32),
    )(starts, xs, We, ber)


_UCHUNK = 32


def _unperm_body(dest_hbm, outs_hbm, out_hbm, dest_v, rows0_v, rows1_v,
                 sem0, sem1):
    c = lax.axis_index("c")
    s = lax.axis_index("s")
    w = 16 * c + s
    base = _TPW * w
    nck = _TPW // _UCHUNK
    pltpu.sync_copy(dest_hbm.at[pl.ds(base, _TPW)], dest_v)
    rows = (rows0_v, rows1_v)
    sems = (sem0, sem1)
    cps = [None] * nck
    cps[0] = pltpu.async_copy(
        outs_hbm.at[dest_v.at[pl.ds(0, _UCHUNK)]], rows[0], sems[0])
    for k in range(nck):
        if k + 1 < nck:
            cps[k + 1] = pltpu.async_copy(
                outs_hbm.at[dest_v.at[pl.ds((k + 1) * _UCHUNK, _UCHUNK)]],
                rows[(k + 1) % 2], sems[(k + 1) % 2])
        cps[k].wait()
        pltpu.sync_copy(rows[k % 2],
                        out_hbm.at[pl.ds(base + k * _UCHUNK, _UCHUNK)])


def _unperm(dest, outs):
    mesh = plsc.VectorSubcoreMesh(core_axis_name="c", subcore_axis_name="s")
    f = pl.kernel(
        _unperm_body,
        out_type=jax.ShapeDtypeStruct((_TOKENS, _DOUT), jnp.float32),
        mesh=mesh,
        scratch_types=[
            pltpu.VMEM((_TPW,), jnp.int32),
            pltpu.VMEM((_UCHUNK, _DOUT), jnp.float32),
            pltpu.VMEM((_UCHUNK, _DOUT), jnp.float32),
            pltpu.SemaphoreType.DMA,
            pltpu.SemaphoreType.DMA,
        ],
    )
    return f(dest, outs)


@jax.jit
def kernel(x, We, be, W1, b1, W2, b2):
    b1r = b1.reshape(1, _H)
    b2r = b2.reshape(1, _E)
    ber = be.reshape(1, _E * _DOUT)
    web = We.astype(jnp.bfloat16)
    ids = _router(x, W1, b1r, W2, b2r)
    dest, xs, starts = _dispatch(ids, x)
    outs = _segmm(starts, xs, web, ber)
    return _unperm(dest, outs)


# speedup vs baseline: 1.0454x; 1.0454x over previous
import functools

import jax
import jax.numpy as jnp
from jax import lax
from jax.experimental import pallas as pl
from jax.experimental.pallas import tpu as pltpu
from jax.experimental.pallas import tpu_sc as plsc

_TOKENS, _DIN, _DOUT, _E = 4096, 768, 1024, 8
_H = 1536
_BLK = 256
_NW = 32
_TPW = _TOKENS // _NW


def _dot(a, b):
    return jax.lax.dot_general(a, b, (((1,), (0,)), ((), ())),
                               preferred_element_type=jnp.float32)


def _router_body(x_ref, w1_ref, b1_ref, w2_ref, b2_ref, o_ref):
    x = x_ref[...]
    h = jnp.maximum(_dot(x, w1_ref[...]) + b1_ref[...], 0.0)
    s = _dot(h, w2_ref[...]) + b2_ref[...]
    o_ref[...] = jnp.argmax(s, axis=1).astype(jnp.int32)


def _router(x, W1, b1r, w2r, b2r):
    nb = _TOKENS // _BLK
    return pl.pallas_call(
        _router_body,
        grid=(nb,),
        in_specs=[
            pl.BlockSpec((_BLK, _DIN), lambda b: (b, 0)),
            pl.BlockSpec((_DIN, _H), lambda b: (0, 0)),
            pl.BlockSpec((1, _H), lambda b: (0, 0)),
            pl.BlockSpec((_H, _E), lambda b: (0, 0)),
            pl.BlockSpec((1, _E), lambda b: (0, 0)),
        ],
        out_specs=pl.BlockSpec((_BLK,), lambda b: (b,)),
        out_shape=jax.ShapeDtypeStruct((_TOKENS,), jnp.int32),
    )(x, W1, b1r, w2r, b2r)


_GDN = lax.GatherDimensionNumbers(
    offset_dims=(), collapsed_slice_dims=(0,), start_index_map=(0,))


def _take16(v, idx):
    return lax.gather(v, idx[:, None], _GDN, (1,),
                      mode=lax.GatherScatterMode.PROMISE_IN_BOUNDS)


def _splat16(v, j):
    return _take16(v, jnp.full((16,), j, jnp.int32))


def _cumsum16(v, lanes):
    for sh in (1, 2, 4, 8):
        idx = jnp.maximum(lanes - sh, 0)
        v = v + jnp.where(lanes >= sh, _take16(v, idx), 0)
    return v


def _count_slice(ids_v, base, lanes):
    cnts = [jnp.zeros((16,), jnp.int32) for _ in range(_E)]
    for k in range(8):
        v = ids_v[pl.ds(base + 16 * k, 16)]
        for e in range(_E):
            mi = jnp.where(v == e, 1, 0)
            cnts[e] = cnts[e] + _splat16(_cumsum16(mi, lanes), 15)
    vec = jnp.zeros((16,), jnp.int32)
    for e in range(_E):
        vec = jnp.where(lanes == e, cnts[e], vec)
    return vec


def _dispatch_body(ids_hbm, x_hbm, dest_hbm, xs_hbm, starts_hbm,
                   ids2_v, idsw_v, cnt2_v, table_v, dest_v, starts_v,
                   xrows_v, shared_cnt, sem, sem_x, sem_w):
    c = lax.axis_index("c")
    s = lax.axis_index("s")
    w = 16 * c + s
    lanes = lax.iota(jnp.int32, 16)

    cp_x = pltpu.async_copy(x_hbm.at[pl.ds(_TPW * w, _TPW)], xrows_v, sem_x)
    cp_w = pltpu.async_copy(ids_hbm.at[pl.ds(_TPW * w, _TPW)], idsw_v, sem_w)

    pltpu.sync_copy(ids_hbm.at[pl.ds(256 * s, 256)], ids2_v)
    for j in range(2):
        cnt2_v[pl.ds(16 * j, 16)] = _count_slice(ids2_v, 128 * j, lanes)
    pltpu.sync_copy(cnt2_v, shared_cnt.at[pl.ds(32 * s, 32)])
    plsc.subcore_barrier()
    pltpu.sync_copy(shared_cnt, table_v)

    totals = jnp.zeros((16,), jnp.int32)
    prior = jnp.zeros((16,), jnp.int32)
    for r in range(_NW):
        row = table_v[pl.ds(16 * r, 16)]
        totals = totals + row
        prior = prior + row * jnp.where(r < w, 1, 0)
    starts_vec = _cumsum16(totals, lanes) - totals
    base_vec = starts_vec + prior

    @pl.when(w == 0)
    def _():
        starts_v[...] = starts_vec
        pltpu.sync_copy(starts_v, starts_hbm)

    cp_w.wait()
    base_run = [_splat16(base_vec, e) for e in range(_E)]
    for k in range(8):
        v = idsw_v[pl.ds(16 * k, 16)]
        dest = jnp.zeros((16,), jnp.int32)
        for e in range(_E):
            mi = jnp.where(v == e, 1, 0)
            incl = _cumsum16(mi, lanes)
            cand = base_run[e] + incl - 1
            dest = dest + mi * (cand - dest)
            base_run[e] = base_run[e] + _splat16(incl, 15)
        dest_v[pl.ds(16 * k, 16)] = dest
    pltpu.sync_copy(dest_v, dest_hbm.at[pl.ds(_TPW * w, _TPW)])

    cp_x.wait()
    pltpu.async_copy(xrows_v, xs_hbm.at[dest_v], sem).wait()


def _dispatch(ids, x):
    mesh = plsc.VectorSubcoreMesh(core_axis_name="c", subcore_axis_name="s")
    f = pl.kernel(
        _dispatch_body,
        out_type=(
            jax.ShapeDtypeStruct((_TOKENS,), jnp.int32),
            jax.ShapeDtypeStruct((_TOKENS, _DIN), jnp.float32),
            jax.ShapeDtypeStruct((16,), jnp.int32),
        ),
        mesh=mesh,
        scratch_types=[
            pltpu.VMEM((256,), jnp.int32),
            pltpu.VMEM((_TPW,), jnp.int32),
            pltpu.VMEM((32,), jnp.int32),
            pltpu.VMEM((16 * _NW,), jnp.int32),
            pltpu.VMEM((_TPW,), jnp.int32),
            pltpu.VMEM((16,), jnp.int32),
            pltpu.VMEM((_TPW, _DIN), jnp.float32),
            pltpu.VMEM_SHARED((16 * _NW,), jnp.int32),
            pltpu.SemaphoreType.DMA,
            pltpu.SemaphoreType.DMA,
            pltpu.SemaphoreType.DMA,
        ],
    )
    return f(ids, x)


def _segmm_body(s_ref, xs_ref, we_ref, be_ref, o_ref):
    b = pl.program_id(0)
    r0 = b * _BLK
    o_ref[...] = jnp.zeros((_BLK, _DOUT), jnp.float32)
    rowid = r0 + jax.lax.broadcasted_iota(jnp.int32, (_BLK, 1), 0)
    for e in range(_E):
        lo = s_ref[e]
        hi = s_ref[e + 1]
        cond = jnp.logical_and(lo < r0 + _BLK, hi > r0)

        @pl.when(cond)
        def _(e=e, lo=lo, hi=hi):
            xb = xs_ref[...].astype(jnp.bfloat16)
            wb = we_ref[:, e * _DOUT:(e + 1) * _DOUT].astype(jnp.bfloat16)
            pe = _dot(xb, wb) + be_ref[:, e * _DOUT:(e + 1) * _DOUT]
            m = jnp.logical_and(rowid >= lo, rowid < hi)
            o_ref[...] += jnp.where(m, pe, 0.0)


def _segmm(starts, xs, We, ber):
    nb = _TOKENS // _BLK
    grid_spec = pltpu.PrefetchScalarGridSpec(
        num_scalar_prefetch=1,
        grid=(nb,),
        in_specs=[
            pl.BlockSpec((_BLK, _DIN), lambda b, s: (b, 0)),
            pl.BlockSpec((_DIN, _E * _DOUT), lambda b, s: (0, 0)),
            pl.BlockSpec((1, _E * _DOUT), lambda b, s: (0, 0)),
        ],
        out_specs=pl.BlockSpec((_BLK, _DOUT), lambda b, s: (b, 0)),
    )
    return pl.pallas_call(
        _segmm_body,
        grid_spec=grid_spec,
        out_shape=jax.ShapeDtypeStruct((_TOKENS, _DOUT), jnp.float32),
    )(starts, xs, We, ber)


_UCHUNK = 32


def _unperm_body(dest_hbm, outs_hbm, out_hbm, dest_v, rows0_v, rows1_v,
                 sem0, sem1):
    c = lax.axis_index("c")
    s = lax.axis_index("s")
    w = 16 * c + s
    base = _TPW * w
    nck = _TPW // _UCHUNK
    pltpu.sync_copy(dest_hbm.at[pl.ds(base, _TPW)], dest_v)
    rows = (rows0_v, rows1_v)
    sems = (sem0, sem1)
    cps = [None] * nck
    cps[0] = pltpu.async_copy(
        outs_hbm.at[dest_v.at[pl.ds(0, _UCHUNK)]], rows[0], sems[0])
    for k in range(nck):
        if k + 1 < nck:
            cps[k + 1] = pltpu.async_copy(
                outs_hbm.at[dest_v.at[pl.ds((k + 1) * _UCHUNK, _UCHUNK)]],
                rows[(k + 1) % 2], sems[(k + 1) % 2])
        cps[k].wait()
        pltpu.sync_copy(rows[k % 2],
                        out_hbm.at[pl.ds(base + k * _UCHUNK, _UCHUNK)])


def _unperm(dest, outs):
    mesh = plsc.VectorSubcoreMesh(core_axis_name="c", subcore_axis_name="s")
    f = pl.kernel(
        _unperm_body,
        out_type=jax.ShapeDtypeStruct((_TOKENS, _DOUT), jnp.float32),
        mesh=mesh,
        scratch_types=[
            pltpu.VMEM((_TPW,), jnp.int32),
            pltpu.VMEM((_UCHUNK, _DOUT), jnp.float32),
            pltpu.VMEM((_UCHUNK, _DOUT), jnp.float32),
            pltpu.SemaphoreType.DMA,
            pltpu.SemaphoreType.DMA,
        ],
    )
    return f(dest, outs)


@jax.jit
def kernel(x, We, be, W1, b1, W2, b2):
    b1r = b1.reshape(1, _H)
    b2r = b2.reshape(1, _E)
    ber = be.reshape(1, _E * _DOUT)
    ids = _router(x, W1, b1r, W2, b2r)
    dest, xs, starts = _dispatch(ids, x)
    outs = _segmm(starts, xs, We, ber)
    return _unperm(dest, outs)
